# baseline (speedup 1.0000x reference)
import functools

import jax
import jax.numpy as jnp
from jax import lax
from jax.experimental import pallas as pl
from jax.experimental.pallas import tpu as pltpu
from jax.experimental.pallas import tpu_sc as plsc

_DIM = 300
_DIMP = 384
_SP = 56
_NW = 32
_BPW = 128


def _gather(table_hbm, tail_hbm, idx_row, asm, sem):
    pltpu.async_copy(
        table_hbm.at[idx_row, pl.ds(0, 256)], asm.at[:, pl.ds(0, 256)], sem
    )
    pltpu.async_copy(tail_hbm.at[idx_row], asm.at[:, pl.ds(256, 128)], sem)


def _wait_gather(table_hbm, tail_hbm, idx_row, asm, sem):
    pltpu.make_async_copy(
        table_hbm.at[idx_row, pl.ds(0, 256)], asm.at[:, pl.ds(0, 256)], sem
    ).wait()
    pltpu.make_async_copy(
        tail_hbm.at[idx_row], asm.at[:, pl.ds(256, 128)], sem
    ).wait()


def _body(idx_hbm, table_hbm, tail_hbm, out_hbm, idx_v, asm0, asm1, sem0, sem1):
    c = lax.axis_index("c")
    s = lax.axis_index("s")
    wid = s * 2 + c
    b0 = wid * _BPW

    pltpu.sync_copy(idx_hbm.at[pl.ds(b0 * _SP, _BPW * _SP)], idx_v)

    asms = (asm0, asm1)
    sems = (sem0, sem1)

    def idx_at(r):
        return idx_v.at[pl.ds(r * _SP, _SP)]

    _gather(table_hbm, tail_hbm, idx_at(0), asm0, sem0)
    _gather(table_hbm, tail_hbm, idx_at(1), asm1, sem1)

    @pl.loop(0, _BPW - 2, step=2)
    def _(r):
        for p in range(2):
            _wait_gather(table_hbm, tail_hbm, idx_at(r + p), asms[p], sems[p])
            pltpu.sync_copy(asms[p], out_hbm.at[b0 + r + p])
            _gather(table_hbm, tail_hbm, idx_at(r + p + 2), asms[p], sems[p])

    for p in range(2):
        r = _BPW - 2 + p
        _wait_gather(table_hbm, tail_hbm, idx_at(r), asms[p], sems[p])
        pltpu.sync_copy(asms[p], out_hbm.at[b0 + r])


@functools.partial(jax.jit, static_argnums=(3,))
def _lookup(idx, table, tail, b):
    mesh = plsc.VectorSubcoreMesh(core_axis_name="c", subcore_axis_name="s")
    f = pl.kernel(
        _body,
        out_type=jax.ShapeDtypeStruct((b, _SP, _DIMP), jnp.float32),
        mesh=mesh,
        scratch_types=[
            pltpu.VMEM((_BPW * _SP,), jnp.int32),
            pltpu.VMEM((_SP, _DIMP), jnp.float32),
            pltpu.VMEM((_SP, _DIMP), jnp.float32),
            pltpu.SemaphoreType.DMA,
            pltpu.SemaphoreType.DMA,
        ],
    )
    return f(idx, table, tail)


def kernel(inp, table):
    b, s = inp.shape
    tail = jnp.pad(
        lax.slice(table, (0, 256), (table.shape[0], _DIM)),
        ((0, 0), (0, 128 - (_DIM - 256))),
    )
    idx = jnp.pad(inp, ((0, 0), (0, _SP - s))).reshape(b * _SP)
    y = _lookup(idx, table, tail, b)
    return y[:, :s, :_DIM]
